# baseline (speedup 1.0000x reference)
import functools

import jax
import jax.numpy as jnp
from jax import lax
from jax.experimental import pallas as pl
from jax.experimental import layout
from jax.experimental.pallas import tpu as pltpu
from jax.experimental.pallas import tpu_sc as plsc

_BATCH = 4096
_SEQ = 200
_DIM = 32
_VOCAB = 1000
_N = _BATCH * _SEQ

_NC = 2
_NS = 16
_NW = _NC * _NS

_ROWS_PER_W = _N // _NW
_NB = 4
_CHUNK = _NB * _SEQ
_IW = 80
_NSUB = _CHUNK // _IW
_NCHUNKS = _ROWS_PER_W // _CHUNK

assert _ROWS_PER_W % _CHUNK == 0


def _body(x_hbm, tab_hbm, pos_hbm, out_hbm, idx_v, stage_v, out_v, pos_v, sem):
  wid = lax.axis_index("s") * _NC + lax.axis_index("c")
  base = wid * _ROWS_PER_W

  pltpu.sync_copy(pos_hbm, pos_v)

  def chunk_body(g, carry):
    off = pl.multiple_of(base + g * _CHUNK, _CHUNK)

    pltpu.sync_copy(x_hbm.at[pl.ds(off, _CHUNK)], idx_v)

    for j in range(_NSUB):
      pltpu.async_copy(
          tab_hbm.at[idx_v.at[pl.ds(j * _IW, _IW)]],
          stage_v.at[pl.ds(j * _IW, _IW)], sem)
    for j in range(_NSUB):
      pltpu.make_async_copy(
          tab_hbm.at[idx_v.at[pl.ds(j * _IW, _IW)]],
          stage_v.at[pl.ds(j * _IW, _IW)], sem).wait()

    def add_body(s, c):
      pv0 = pos_v[s, pl.ds(0, 16)]
      pv1 = pos_v[s, pl.ds(16, 16)]
      for b in range(_NB):
        i = _SEQ * b + s
        out_v[b, s, pl.ds(0, 16)] = stage_v[i, pl.ds(0, 16)] + pv0
        out_v[b, s, pl.ds(16, 16)] = stage_v[i, pl.ds(16, 16)] + pv1
      return c

    lax.fori_loop(0, _SEQ, add_body, 0, unroll=2)

    b0 = pl.multiple_of(off // _SEQ, _NB)
    pltpu.sync_copy(out_v, out_hbm.at[pl.ds(b0, _NB)])
    return carry

  lax.fori_loop(0, _NCHUNKS, chunk_body, 0)


def _embed(x1d, nucleo_emb, pos_emb):
  mesh = plsc.VectorSubcoreMesh(
      core_axis_name="c", subcore_axis_name="s", num_cores=_NC,
      num_subcores=_NS)
  return pl.kernel(
      _body,
      out_type=jax.ShapeDtypeStruct((_BATCH, _SEQ, _DIM), jnp.float32),
      mesh=mesh,
      compiler_params=pltpu.CompilerParams(use_tc_tiling_on_sc=False),
      scratch_types=[
          pltpu.VMEM((_CHUNK,), jnp.int32),
          pltpu.VMEM((_CHUNK, _DIM), jnp.float32),
          pltpu.VMEM((_NB, _SEQ, _DIM), jnp.float32),
          pltpu.VMEM((_SEQ, _DIM), jnp.float32),
          pltpu.SemaphoreType.DMA,
      ],
  )(x1d, nucleo_emb, pos_emb)


def kernel(X, nucleo_emb, pos_emb):
  return _embed(X.reshape(_N), nucleo_emb, pos_emb)
